# baseline (speedup 1.0000x reference)
import math

import jax
import jax.numpy as jnp
import numpy as np
from jax.experimental import pallas as pl
from jax import shard_map
from jax.sharding import Mesh, PartitionSpec as P

_DIM = 2048
_BT = 512
_NC = 4


def _quantize(y, c):
    b3 = (c[3] + c[4]) * 0.5
    b45 = (c[4] + c[5]) * 0.5
    b56 = (c[5] + c[6]) * 0.5
    b67 = (c[6] + c[7]) * 0.5
    m = jnp.abs(y)
    k1 = m > b56
    bs = jnp.where(k1, b67, b45)
    k2 = m > bs
    hi = jnp.where(k1, c[7], c[5])
    lo = jnp.where(k1, c[6], c[4])
    mag = jnp.where(k2, hi, lo)
    return jnp.where(y > b3, mag, -mag)


def _fused_kernel(x_ref, pi_ref, c_ref, out_ref):
    x = x_ref[...]
    norm = jnp.sqrt(jnp.sum(x * x, axis=1, keepdims=True))
    unit = x * (1.0 / (norm + 1e-10))
    c = [c_ref[0, i] for i in range(8)]

    cw = _DIM // _NC
    ys = []
    yhats = []
    for j in range(_NC):
        yj = jnp.dot(unit, pi_ref[:, j * cw:(j + 1) * cw],
                     preferred_element_type=jnp.float32)
        ys.append(yj)
        if j > 0:
            yhats.append(_quantize(ys[j - 1], c))
    yhats.append(_quantize(ys[-1], c))
    y_hat = jnp.concatenate(yhats, axis=1)

    x_hat = jax.lax.dot_general(
        y_hat, pi_ref[...],
        dimension_numbers=(((1,), (1,)), ((), ())),
        preferred_element_type=jnp.float32,
    )
    out_ref[...] = x_hat * norm


def _run_shard(x, Pi, c2d):
    n_tok, dim = x.shape
    grid = (n_tok // _BT,)
    return pl.pallas_call(
        _fused_kernel,
        grid=grid,
        in_specs=[
            pl.BlockSpec((_BT, dim), lambda i: (i, 0)),
            pl.BlockSpec((dim, dim), lambda i: (0, 0)),
            pl.BlockSpec((1, 8), lambda i: (0, 0)),
        ],
        out_specs=pl.BlockSpec((_BT, dim), lambda i: (i, 0)),
        out_shape=jax.ShapeDtypeStruct((n_tok, dim), jnp.float32),
    )(x, Pi, c2d)


def kernel(x, Pi, centroids):
    n_tok, _ = x.shape
    c2d = centroids.reshape(1, -1)
    devs = [d for d in jax.devices() if d.platform == "tpu"]
    n_sh = math.gcd(n_tok // _BT, max(len(devs), 1))
    if n_sh > 1:
        mesh = Mesh(np.asarray(devs[:n_sh]), ("d",))
        sharded = shard_map(
            _run_shard,
            mesh=mesh,
            in_specs=(P("d", None), P(None, None), P(None, None)),
            out_specs=P("d", None),
            check_vma=False,
        )
        return sharded(x, Pi, c2d)
    return _run_shard(x, Pi, c2d)


# speedup vs baseline: 2.9157x; 2.9157x over previous
import jax
import jax.numpy as jnp
from jax.experimental import pallas as pl
from jax.experimental.pallas import tpu as pltpu

_DIM = 2048
_BT = 512
_NC = 4


def _quantize(y, c):
    b3 = (c[3] + c[4]) * 0.5
    b45 = (c[4] + c[5]) * 0.5
    b56 = (c[5] + c[6]) * 0.5
    b67 = (c[6] + c[7]) * 0.5
    m = jnp.abs(y)
    k1 = m > b56
    bs = jnp.where(k1, b67, b45)
    k2 = m > bs
    hi = jnp.where(k1, c[7], c[5])
    lo = jnp.where(k1, c[6], c[4])
    mag = jnp.where(k2, hi, lo)
    return jnp.where(y > b3, mag, -mag)


def _fused_kernel(x_ref, pi_ref, c_ref, out_ref):
    x = x_ref[...]
    norm = jnp.sqrt(jnp.sum(x * x, axis=1, keepdims=True))
    unit = (x * (1.0 / (norm + 1e-10))).astype(jnp.bfloat16)
    pi = pi_ref[...].astype(jnp.bfloat16)
    c = [c_ref[0, i] for i in range(8)]

    cw = _DIM // _NC
    ys = []
    yhats = []
    for j in range(_NC):
        yj = jnp.dot(unit, pi[:, j * cw:(j + 1) * cw],
                     preferred_element_type=jnp.float32)
        ys.append(yj)
        if j > 0:
            yhats.append(_quantize(ys[j - 1], c).astype(jnp.bfloat16))
    yhats.append(_quantize(ys[-1], c).astype(jnp.bfloat16))
    y_hat = jnp.concatenate(yhats, axis=1)

    x_hat = jax.lax.dot_general(
        y_hat, pi,
        dimension_numbers=(((1,), (1,)), ((), ())),
        preferred_element_type=jnp.float32,
    )
    out_ref[...] = x_hat * norm


def kernel(x, Pi, centroids):
    n_tok, dim = x.shape
    c2d = centroids.reshape(1, -1)
    grid = (n_tok // _BT,)
    return pl.pallas_call(
        _fused_kernel,
        grid=grid,
        in_specs=[
            pl.BlockSpec((_BT, dim), lambda i: (i, 0)),
            pl.BlockSpec((dim, dim), lambda i: (0, 0)),
            pl.BlockSpec((1, 8), lambda i: (0, 0)),
        ],
        out_specs=pl.BlockSpec((_BT, dim), lambda i: (i, 0)),
        out_shape=jax.ShapeDtypeStruct((n_tok, dim), jnp.float32),
        compiler_params=pltpu.CompilerParams(
            dimension_semantics=("arbitrary",),
        ),
    )(x, Pi, c2d)
